# baseline (speedup 1.0000x reference)
import functools

import jax
import jax.numpy as jnp
from jax import lax
from jax.experimental import pallas as pl
from jax.experimental.pallas import tpu as pltpu
from jax.experimental.pallas import tpu_sc as plsc

B, T, H, D = 1024, 20, 26, 32
NC, NS = 2, 16
NW = NC * NS
NROWS = H * T
RPW = (NROWS + NW - 1) // NW
CH = 128
CPR = B // CH


def _tc_shift_body(ids_ref, offs_ref, out_ref):
    out_ref[...] = ids_ref[...] + offs_ref[...]


_tc_shift = pl.pallas_call(
    _tc_shift_body,
    out_shape=jax.ShapeDtypeStruct((H, T, B), jnp.int32),
)


def _sc_body(ids_hbm, table_hbm, out_hbm, row_v, rows0_v, rows1_v, trans_v,
             sem_g, sem_o):
    wid = lax.axis_index("s") * NC + lax.axis_index("c")
    iota = lax.broadcasted_iota(jnp.int32, (16,), 0)

    def fire(c, buf):
        pltpu.async_copy(table_hbm.at[row_v.at[pl.ds(c * CH, CH)]], buf,
                         sem_g)

    def process(c, buf, t, h, first):
        pltpu.make_async_copy(table_hbm.at[row_v.at[pl.ds(c * CH, CH)]],
                              buf, sem_g).wait()
        b0 = c * CH
        dst = out_hbm.at[t, h, :, pl.ds(b0, CH)]

        @pl.when(jnp.logical_not(first))
        def _():
            pltpu.make_async_copy(trans_v, dst, sem_o).wait()

        def tr(j0, carry):
            rowi = iota + j0 * 16
            for d in range(D):
                v = plsc.load_gather(buf, [rowi, jnp.full((16,), d,
                                                          jnp.int32)])
                trans_v[d, pl.ds(j0 * 16, 16)] = v
            return carry

        lax.fori_loop(0, CH // 16, tr, 0)
        pltpu.async_copy(trans_v, dst, sem_o)

    def row_body(k, carry):
        r = k * NW + wid

        @pl.when(r < NROWS)
        def _():
            h = lax.div(r, T)
            t = r - h * T
            pltpu.sync_copy(ids_hbm.at[h, t], row_v)
            fire(0, rows0_v)

            def pair(p, c2):
                c0 = p * 2
                fire(c0 + 1, rows1_v)
                process(c0, rows0_v, t, h, jnp.logical_and(k == 0, c0 == 0))

                @pl.when(c0 + 2 < CPR)
                def _():
                    fire(c0 + 2, rows0_v)

                process(c0 + 1, rows1_v, t, h, False)
                return c2

            lax.fori_loop(0, CPR // 2, pair, 0)

        return carry

    lax.fori_loop(0, RPW, row_body, 0)

    pltpu.make_async_copy(trans_v, out_hbm.at[0, 0, :, pl.ds(0, CH)],
                          sem_o).wait()


@functools.partial(
    pl.kernel,
    out_type=jax.ShapeDtypeStruct((T, H, D, B), jnp.float32),
    mesh=plsc.VectorSubcoreMesh(core_axis_name="c", subcore_axis_name="s"),
    scratch_types=[
        pltpu.VMEM((B,), jnp.int32),
        pltpu.VMEM((CH, D), jnp.float32),
        pltpu.VMEM((CH, D), jnp.float32),
        pltpu.VMEM((D, CH), jnp.float32),
        pltpu.SemaphoreType.DMA,
        pltpu.SemaphoreType.DMA,
    ],
    compiler_params=pltpu.CompilerParams(use_tc_tiling_on_sc=False,
                                         needs_layout_passes=False),
)
def _sc_gather(ids_hbm, table_hbm, out_hbm, row_v, rows0_v, rows1_v,
               trans_v, sem_g, sem_o):
    _sc_body(ids_hbm, table_hbm, out_hbm, row_v, rows0_v, rows1_v, trans_v,
             sem_g, sem_o)


def kernel(input_ids, table, offsets):
    ids_htb = input_ids.transpose(2, 1, 0).astype(jnp.int32)
    offs3 = offsets.astype(jnp.int32).reshape(H, 1, 1)
    shifted = _tc_shift(ids_htb, offs3)
    out_t = _sc_gather(shifted, table)
    return out_t.transpose(3, 0, 1, 2)


# speedup vs baseline: 1.1014x; 1.1014x over previous
import functools

import jax
import jax.numpy as jnp
from jax import lax
from jax.experimental import pallas as pl
from jax.experimental.pallas import tpu as pltpu
from jax.experimental.pallas import tpu_sc as plsc

B, T, H, D = 1024, 20, 26, 32
NC, NS = 2, 16
NW = NC * NS
RPW = B // NW


def _sc_body(ids_hbm, pat_hbm, table_hbm, out_hbm, idx0_v, idx1_v,
             rows0_v, rows1_v, pat_v, sem_g, sem_o):
    wid = lax.axis_index("s") * NC + lax.axis_index("c")

    pltpu.sync_copy(pat_hbm, pat_v)
    pa = pat_v[pl.ds(0, 16)]
    pb = pat_v[pl.ds(16, 16)]

    def stage(b, idx_v):
        pltpu.sync_copy(ids_hbm.at[b], idx_v)

        def add_row(t, carry):
            idx_v[t, pl.ds(0, 16)] = idx_v[t, pl.ds(0, 16)] + pa
            idx_v[t, pl.ds(10, 16)] = idx_v[t, pl.ds(10, 16)] + pb
            return carry

        lax.fori_loop(0, T, add_row, 0, unroll=4)

    def fire(idx_v, buf):
        for t in range(T):
            pltpu.async_copy(table_hbm.at[idx_v.at[t]], buf.at[t], sem_g)

    def drain(idx_v, buf):
        for t in range(T):
            pltpu.make_async_copy(table_hbm.at[idx_v.at[t]], buf.at[t],
                                  sem_g).wait()

    def out_copy(b, buf):
        pltpu.async_copy(buf, out_hbm.at[b], sem_o)

    def out_wait(b, buf):
        pltpu.make_async_copy(buf, out_hbm.at[b], sem_o).wait()

    b0 = wid * RPW
    stage(b0, idx0_v)
    fire(idx0_v, rows0_v)

    def pair(p, carry):
        b = b0 + p * 2
        stage(b + 1, idx1_v)
        fire(idx1_v, rows1_v)
        drain(idx0_v, rows0_v)
        out_copy(b, rows0_v)

        @pl.when(p + 1 < RPW // 2)
        def _():
            stage(b + 2, idx0_v)
            out_wait(b, rows0_v)
            fire(idx0_v, rows0_v)

        drain(idx1_v, rows1_v)
        out_copy(b + 1, rows1_v)

        @pl.when(p + 1 < RPW // 2)
        def _():
            out_wait(b + 1, rows1_v)

        return carry

    lax.fori_loop(0, RPW // 2, pair, 0)
    out_wait(b0 + RPW - 2, rows0_v)
    out_wait(b0 + RPW - 1, rows1_v)


@functools.partial(
    pl.kernel,
    out_type=jax.ShapeDtypeStruct((B, T, H, D), jnp.float32),
    mesh=plsc.VectorSubcoreMesh(core_axis_name="c", subcore_axis_name="s"),
    scratch_types=[
        pltpu.VMEM((T, H), jnp.int32),
        pltpu.VMEM((T, H), jnp.int32),
        pltpu.VMEM((T, H, D), jnp.float32),
        pltpu.VMEM((T, H, D), jnp.float32),
        pltpu.VMEM((32,), jnp.int32),
        pltpu.SemaphoreType.DMA,
        pltpu.SemaphoreType.DMA,
    ],
    compiler_params=pltpu.CompilerParams(use_tc_tiling_on_sc=False),
)
def _sc_gather(ids_hbm, pat_hbm, table_hbm, out_hbm, idx0_v, idx1_v,
               rows0_v, rows1_v, pat_v, sem_g, sem_o):
    _sc_body(ids_hbm, pat_hbm, table_hbm, out_hbm, idx0_v, idx1_v,
             rows0_v, rows1_v, pat_v, sem_g, sem_o)


def kernel(input_ids, table, offsets):
    offs = offsets.astype(jnp.int32)
    pat = jnp.concatenate(
        [offs[0:16], jnp.zeros((6,), jnp.int32), offs[16:26]])
    return _sc_gather(input_ids, pat, table)
